# baseline (speedup 1.0000x reference)
import jax
import jax.numpy as jnp
from jax import lax
from jax.experimental import pallas as pl
from jax.experimental.pallas import tpu as pltpu
from jax.experimental.pallas import tpu_sc as plsc

T = 2048
S = 2048
NHK = 10
ROWS = 256
NW = 32
TPW = T // NW
CH = 4
NCH = TPW // CH
D = 1024


def _topk_body(rel_ref, idx_ref, w_ref):
    y = rel_ref[...]
    y = y * y
    ql = lax.broadcasted_iota(jnp.int32, (128, 64), 0)
    qm = lax.broadcasted_iota(jnp.int32, (128, 64), 1)
    q = jnp.where(jnp.right_shift(ql, 1) == qm, 1.0, 0.0).astype(jnp.float32)
    sq2 = lax.dot_general(
        y, q, (((1,), (0,)), ((), ())),
        precision=lax.Precision.HIGHEST,
        preferred_element_type=jnp.float32)
    sq3 = sq2.reshape(ROWS, 32, 64)
    sq = jnp.concatenate([sq3[:, :16, :], sq3[:, 16:, :]], axis=2)
    gi = lax.broadcasted_iota(jnp.int32, (ROWS, 16, 128), 1)
    mi = lax.broadcasted_iota(jnp.int32, (ROWS, 16, 128), 2)
    iot = (64 * gi + 1024 * jnp.right_shift(mi, 6)
           + jnp.bitwise_and(mi, 63))
    big = jnp.float32(jnp.inf)
    ws = []
    idxs = []
    wsum = jnp.zeros((ROWS, 1), jnp.float32)
    for j in range(NHK):
        m1 = jnp.min(sq, axis=2, keepdims=True)
        mn = jnp.min(m1, axis=1, keepdims=True)
        a1 = jnp.min(jnp.where(sq == mn, iot, S), axis=2, keepdims=True)
        am = jnp.min(a1, axis=1, keepdims=True)
        sq = jnp.where(iot == am, big, sq)
        wj = 1.0 / (jnp.sqrt(mn.reshape(ROWS, 1) + 1e-12) + 1e-10)
        wsum = wsum + wj
        ws.append(wj)
        idxs.append(am.reshape(ROWS, 1))
    idx_ref[...] = jnp.concatenate(idxs, axis=1)
    w_ref[...] = jnp.concatenate(ws, axis=1) / wsum


def _topk_weights(rel_r):
    return pl.pallas_call(
        _topk_body,
        grid=(T // ROWS,),
        in_specs=[pl.BlockSpec((ROWS * 32, 128), lambda i: (i, 0))],
        out_specs=[
            pl.BlockSpec((ROWS, NHK), lambda i: (i, 0)),
            pl.BlockSpec((ROWS, NHK), lambda i: (i, 0)),
        ],
        out_shape=[
            jax.ShapeDtypeStruct((T, NHK), jnp.int32),
            jax.ShapeDtypeStruct((T, NHK), jnp.float32),
        ],
    )(rel_r)


def _sc_body(xt_hbm, idxf_hbm, wf_hbm, out_hbm, idx_all, w_all,
             rows_a, rows_b, acc_a, acc_b,
             sem_i, sem_w, sem_ga, sem_gb, sem_oa, sem_ob):
    wid = lax.axis_index("s") * 2 + lax.axis_index("c")
    t_base = pl.multiple_of(wid * TPW, TPW)
    f_base = pl.multiple_of(t_base * NHK, TPW * NHK)
    pltpu.async_copy(idxf_hbm.at[pl.ds(f_base, TPW * NHK)], idx_all,
                     sem_i).wait()
    pltpu.async_copy(wf_hbm.at[pl.ds(f_base * 16, TPW * NHK * 16)], w_all,
                     sem_w).wait()

    rows = (rows_a, rows_b)
    accs = (acc_a, acc_b)
    gsem = (sem_ga, sem_gb)
    osem = (sem_oa, sem_ob)

    def start_gather(c, b):
        pltpu.async_copy(
            xt_hbm.at[idx_all.at[pl.ds(c * (CH * NHK), CH * NHK)]],
            rows[b], gsem[b])

    start_gather(0, 0)
    start_gather(1, 1)

    def pair_body(p, carry):
        for b in range(2):
            c = p * 2 + b
            rows_v = rows[b]
            acc_v = accs[b]
            @pl.when(p >= 1)
            def _():
                pltpu.make_async_copy(acc_v, out_hbm.at[pl.ds(0, CH)],
                                      osem[b]).wait()

            pltpu.make_async_copy(
                xt_hbm.at[idx_all.at[pl.ds(c * (CH * NHK), CH * NHK)]],
                rows_v, gsem[b]).wait()

            wo = pl.multiple_of(c * (CH * NHK * 16), CH * NHK * 16)
            for t in range(CH):
                wsp = [w_all[pl.ds(wo + (t * NHK + j) * 16, 16)]
                       for j in range(NHK)]

                def k_body(k, carry2, _t=t, _wsp=wsp, _rv=rows_v, _av=acc_v):
                    for u in range(4):
                        o = pl.multiple_of(k * 64, 64) + u * 16
                        acc = _wsp[0] * _rv[_t * NHK, pl.ds(o, 16)]
                        for j in range(1, NHK):
                            acc = acc + _wsp[j] * _rv[_t * NHK + j,
                                                      pl.ds(o, 16)]
                        _av[_t, pl.ds(o, 16)] = acc
                    return carry2

                lax.fori_loop(0, D // 64, k_body, 0)

            t0 = t_base + c * CH
            pltpu.async_copy(acc_v, out_hbm.at[pl.ds(t0, CH)], osem[b])

            @pl.when(p < NCH // 2 - 1)
            def _():
                start_gather(c + 2, b)

        return carry

    lax.fori_loop(0, NCH // 2, pair_body, 0)
    pltpu.make_async_copy(acc_a, out_hbm.at[pl.ds(0, CH)], sem_oa).wait()
    pltpu.make_async_copy(acc_b, out_hbm.at[pl.ds(0, CH)], sem_ob).wait()


def _sc_interp(xt, idx_flat, w_flat):
    mesh = plsc.VectorSubcoreMesh(core_axis_name="c", subcore_axis_name="s")
    f = pl.kernel(
        _sc_body,
        out_type=jax.ShapeDtypeStruct((T, D), jnp.float32),
        mesh=mesh,
        scratch_types=[
            pltpu.VMEM((TPW * NHK,), jnp.int32),
            pltpu.VMEM((TPW * NHK * 16,), jnp.float32),
            pltpu.VMEM((CH * NHK, D), jnp.float32),
            pltpu.VMEM((CH * NHK, D), jnp.float32),
            pltpu.VMEM((CH, D), jnp.float32),
            pltpu.VMEM((CH, D), jnp.float32),
            pltpu.SemaphoreType.DMA,
            pltpu.SemaphoreType.DMA,
            pltpu.SemaphoreType.DMA,
            pltpu.SemaphoreType.DMA,
            pltpu.SemaphoreType.DMA,
            pltpu.SemaphoreType.DMA,
        ],
    )
    return f(xt, idx_flat, w_flat)


def kernel(x, rel_target_source):
    B, Sx, d = x.shape
    rel_r = rel_target_source.reshape(T * 32, 128)
    idx, w = _topk_weights(rel_r)
    xt = jnp.transpose(x, (1, 0, 2)).reshape(S, B * d)
    wb = jnp.broadcast_to(w.reshape(T * NHK, 1), (T * NHK, 16)).reshape(-1)
    out2d = _sc_interp(xt, idx.reshape(-1), wb)
    return jnp.transpose(out2d.reshape(T, B, d), (1, 0, 2))


# speedup vs baseline: 33.2911x; 33.2911x over previous
import jax
import jax.numpy as jnp
from jax import lax
from jax.experimental import pallas as pl
from jax.experimental.pallas import tpu as pltpu
from jax.experimental.pallas import tpu_sc as plsc

T = 2048
S = 2048
NHK = 10
ROWS = 256
NW = 32
TPW = T // NW
CH = 8
NCH = TPW // CH
D = 1024
DW = D // 2


def _topk_body(rel_ref, idx_ref, w_ref):
    r0 = rel_ref[0]
    r1 = rel_ref[1]
    sq = r0 * r0 + r1 * r1
    iot = lax.broadcasted_iota(jnp.int32, (ROWS, S), 1)
    big = jnp.float32(jnp.inf)
    ws = []
    idxs = []
    wsum = jnp.zeros((ROWS, 1), jnp.float32)
    for j in range(NHK):
        mn = jnp.min(sq, axis=1, keepdims=True)
        am = jnp.min(jnp.where(sq == mn, iot, S), axis=1, keepdims=True)
        sq = jnp.where(iot == am, big, sq)
        wj = 1.0 / (jnp.sqrt(mn + 1e-12) + 1e-10)
        wsum = wsum + wj
        ws.append(wj)
        idxs.append(am)
    idx_ref[...] = jnp.concatenate(idxs, axis=1)
    w_ref[...] = jnp.concatenate(ws, axis=1) / wsum


def _topk_weights(rel_m):
    return pl.pallas_call(
        _topk_body,
        grid=(T // ROWS,),
        in_specs=[pl.BlockSpec((2, ROWS, S), lambda i: (0, i, 0))],
        out_specs=[
            pl.BlockSpec((ROWS, NHK), lambda i: (i, 0)),
            pl.BlockSpec((ROWS, NHK), lambda i: (i, 0)),
        ],
        out_shape=[
            jax.ShapeDtypeStruct((T, NHK), jnp.int32),
            jax.ShapeDtypeStruct((T, NHK), jnp.float32),
        ],
    )(rel_m)


def _sc_body(xt_hbm, idxf_hbm, wf_hbm, out_hbm, idx_all, w_all,
             rows_a, rows_b, acc_a, acc_b,
             sem_i, sem_w, sem_ga, sem_gb, sem_oa, sem_ob):
    wid = lax.axis_index("s") * 2 + lax.axis_index("c")
    t_base = pl.multiple_of(wid * TPW, TPW)
    f_base = pl.multiple_of(t_base * NHK, TPW * NHK)
    pltpu.async_copy(idxf_hbm.at[pl.ds(f_base, TPW * NHK)], idx_all,
                     sem_i).wait()
    pltpu.async_copy(wf_hbm.at[pl.ds(f_base * 16, TPW * NHK * 16)], w_all,
                     sem_w).wait()

    rows = (rows_a, rows_b)
    accs = (acc_a, acc_b)
    gsem = (sem_ga, sem_gb)
    osem = (sem_oa, sem_ob)

    def start_gather(c, b):
        pltpu.async_copy(
            xt_hbm.at[idx_all.at[pl.ds(c * (CH * NHK), CH * NHK)]],
            rows[b], gsem[b])

    start_gather(0, 0)
    start_gather(1, 1)

    def pair_body(p, carry):
        for b in range(2):
            c = p * 2 + b
            rows_v = rows[b]
            acc_v = accs[b]
            @pl.when(p >= 1)
            def _():
                pltpu.make_async_copy(acc_v, out_hbm.at[pl.ds(0, CH)],
                                      osem[b]).wait()

            pltpu.make_async_copy(
                xt_hbm.at[idx_all.at[pl.ds(c * (CH * NHK), CH * NHK)]],
                rows_v, gsem[b]).wait()

            wo = pl.multiple_of(c * (CH * NHK * 16), CH * NHK * 16)
            for t in range(CH):
                wsp = [w_all[pl.ds(wo + (t * NHK + j) * 16, 16)]
                       for j in range(NHK)]

                def k_body(k, carry2, _t=t, _wsp=wsp, _rv=rows_v, _av=acc_v):
                    for u in range(2):
                        ow = pl.multiple_of(k * 32, 32) + u * 16
                        oa = pl.multiple_of(k * 64, 64) + u * 32
                        acc_a = None
                        acc_b = None
                        for j in range(NHK):
                            wv = _rv[_t * NHK + j, pl.ds(ow, 16)]
                            aj = lax.bitcast_convert_type(
                                jnp.left_shift(wv, 16), jnp.float32)
                            bj = lax.bitcast_convert_type(
                                jnp.bitwise_and(wv, jnp.int32(-65536)),
                                jnp.float32)
                            if j == 0:
                                acc_a = _wsp[j] * aj
                                acc_b = _wsp[j] * bj
                            else:
                                acc_a = acc_a + _wsp[j] * aj
                                acc_b = acc_b + _wsp[j] * bj
                        _av[_t, pl.ds(oa, 16)] = acc_a
                        _av[_t, pl.ds(oa + 16, 16)] = acc_b
                    return carry2

                lax.fori_loop(0, DW // 32, k_body, 0)

            t0 = t_base + c * CH
            pltpu.async_copy(acc_v, out_hbm.at[pl.ds(t0, CH)], osem[b])

            @pl.when(p < NCH // 2 - 1)
            def _():
                start_gather(c + 2, b)

        return carry

    lax.fori_loop(0, NCH // 2, pair_body, 0)
    pltpu.make_async_copy(acc_a, out_hbm.at[pl.ds(0, CH)], sem_oa).wait()
    pltpu.make_async_copy(acc_b, out_hbm.at[pl.ds(0, CH)], sem_ob).wait()


def _sc_interp(xt, idx_flat, w_flat):
    mesh = plsc.VectorSubcoreMesh(core_axis_name="c", subcore_axis_name="s")
    f = pl.kernel(
        _sc_body,
        out_type=jax.ShapeDtypeStruct((T, D), jnp.float32),
        mesh=mesh,
        scratch_types=[
            pltpu.VMEM((TPW * NHK,), jnp.int32),
            pltpu.VMEM((TPW * NHK * 16,), jnp.float32),
            pltpu.VMEM((CH * NHK, DW), jnp.int32),
            pltpu.VMEM((CH * NHK, DW), jnp.int32),
            pltpu.VMEM((CH, D), jnp.float32),
            pltpu.VMEM((CH, D), jnp.float32),
            pltpu.SemaphoreType.DMA,
            pltpu.SemaphoreType.DMA,
            pltpu.SemaphoreType.DMA,
            pltpu.SemaphoreType.DMA,
            pltpu.SemaphoreType.DMA,
            pltpu.SemaphoreType.DMA,
        ],
    )
    return f(xt, idx_flat, w_flat)


def kernel(x, rel_target_source):
    B, Sx, d = x.shape
    rel_m = jnp.moveaxis(rel_target_source, -1, 0)
    idx, w = _topk_weights(rel_m)
    xt = jnp.transpose(x, (1, 0, 2)).reshape(S, B * d)
    xt_bf = (xt.reshape(S, 32, 2, 16).swapaxes(2, 3).reshape(S, D)
             .astype(jnp.bfloat16))
    xt_i32 = lax.bitcast_convert_type(xt_bf.reshape(S, DW, 2), jnp.int32)
    wb = jnp.broadcast_to(w.reshape(T * NHK, 1), (T * NHK, 16)).reshape(-1)
    out2d = _sc_interp(xt_i32, idx.reshape(-1), wb)
    return jnp.transpose(out2d.reshape(T, B, d), (1, 0, 2))
